# baseline (speedup 1.0000x reference)
import functools

import jax
import jax.numpy as jnp
from jax import lax
from jax.experimental import pallas as pl
from jax.experimental.pallas import tpu as pltpu
import jax.experimental.pallas.tpu_sc as plsc

B, F, V, D = 16384, 26, 100000, 32
NC, NS = 2, 16
NW = NC * NS
BPW = B // NW
IDX_CH = 128
CH = BPW // IDX_CH
VB = 12800
NVB = (V + VB - 1) // VB
FP = NVB * VB
ECOLS = F * 128
UCOLS = 13 * D
BB = 1024


def _relayout_body(x0_ref, x1_ref, x2_ref, x3_ref, o_ref):
    xs = jnp.concatenate(
        [x0_ref[0], x1_ref[0], x2_ref[0], x3_ref[0]], axis=0)
    r = lax.broadcasted_iota(jnp.int32, (128, 128), 0)
    c = lax.broadcasted_iota(jnp.int32, (128, 128), 1)
    eye = jnp.where(r == c, 1.0, 0.0).astype(jnp.float32)
    o_ref[...] = lax.dot_general(
        xs, eye, (((0,), (0,)), ((), ())),
        preferred_element_type=jnp.float32)


def _relayout(tables):
    tt = jnp.transpose(tables, (0, 2, 1))
    mk = lambda t: pl.BlockSpec(
        (1, D, VB), lambda g, v, t=t: (jnp.minimum(4 * g + t, 12), 0, v))
    return pl.pallas_call(
        _relayout_body,
        grid=(4, NVB),
        in_specs=[mk(0), mk(1), mk(2), mk(3)],
        out_specs=pl.BlockSpec((VB, 128), lambda g, v: (g * NVB + v, 0)),
        out_shape=jax.ShapeDtypeStruct((4 * FP, 128), jnp.float32),
        compiler_params=pltpu.CompilerParams(
            fuse_transposed_lhs_in_matmul=True),
    )(tt, tt, tt, tt)


def _sc_gather(xq, ptab):
    mesh = plsc.VectorSubcoreMesh(
        core_axis_name="c", subcore_axis_name="s",
        num_cores=NC, num_subcores=NS)

    NBUF = 4
    LAG = 2

    @functools.partial(
        pl.kernel,
        out_type=jax.ShapeDtypeStruct((B, 13 * 128), jnp.float32),
        mesh=mesh,
        scratch_types=[
            pltpu.VMEM((2, CH, IDX_CH), jnp.int32),
            pltpu.VMEM((NBUF, IDX_CH, 128), jnp.float32),
            pltpu.SemaphoreType.DMA,
            pltpu.SemaphoreType.DMA,
        ],
    )
    def k(xq_hbm, ptab_hbm, out_hbm, idx_v, rows_v, gsem, wsem):
        wid = lax.axis_index("s") * NC + lax.axis_index("c")
        base = wid * BPW
        nchk = 13 * CH
        gh = [None] * nchk
        wh = [None] * nchk

        def start_gather(kk):
            t, j = kk // CH, kk % CH
            if j == 0:
                pltpu.sync_copy(xq_hbm.at[t, wid], idx_v.at[t % 2])
            if kk >= NBUF:
                wh[kk - NBUF].wait()
            gh[kk] = pltpu.async_copy(
                ptab_hbm.at[idx_v.at[t % 2, j]], rows_v.at[kk % NBUF], gsem)

        def start_write(kk):
            t, j = kk // CH, kk % CH
            gh[kk].wait()
            wh[kk] = pltpu.async_copy(
                rows_v.at[kk % NBUF],
                out_hbm.at[pl.ds(base + j * IDX_CH, IDX_CH),
                           pl.ds(128 * t, 128)], wsem)

        for kk in range(nchk):
            start_gather(kk)
            if kk >= LAG:
                start_write(kk - LAG)
        for kk in range(nchk - LAG, nchk):
            start_write(kk)
        for kk in range(nchk - NBUF, nchk):
            wh[kk].wait()

    return k(xq, ptab)


def _tc_body(embu_ref, embi_ref, uW0_ref, uW1_ref, uW2_ref, ub0_ref, ub1_ref, ub2_ref,
             iW0_ref, iW1_ref, iW2_ref, ib0_ref, ib1_ref, ib2_ref,
             out_ref, wc_ref, bc_ref):
    @pl.when(pl.program_id(0) == 0)
    def _():
        f32 = jnp.float32
        wu = jnp.dot(uW0_ref[...], jnp.dot(uW1_ref[...], uW2_ref[...],
                                           preferred_element_type=f32),
                     preferred_element_type=f32)
        wv = jnp.dot(iW0_ref[...], jnp.dot(iW1_ref[...], iW2_ref[...],
                                           preferred_element_type=f32),
                     preferred_element_type=f32)
        wc_ref[...] = jnp.zeros((ECOLS, 128), f32)
        for t in range(13):
            ro = 32 * (t % 4)
            wc_ref[128 * t + ro:128 * t + ro + D, 0:64] = wu[D * t:D * t + D, :]
            wc_ref[128 * (13 + t) + ro:128 * (13 + t) + ro + D, 64:128] = \
                wv[D * t:D * t + D, :]
        bu = jnp.dot(jnp.dot(ub0_ref[...].reshape(1, 256), uW1_ref[...],
                             preferred_element_type=f32)
                     + ub1_ref[...].reshape(1, 128),
                     uW2_ref[...], preferred_element_type=f32) \
            + ub2_ref[...].reshape(1, 64)
        bv = jnp.dot(jnp.dot(ib0_ref[...].reshape(1, 256), iW1_ref[...],
                             preferred_element_type=f32)
                     + ib1_ref[...].reshape(1, 128),
                     iW2_ref[...], preferred_element_type=f32) \
            + ib2_ref[...].reshape(1, 64)
        bc_ref[0:1, 0:64] = bu
        bc_ref[0:1, 64:128] = bv

    emb = jnp.concatenate([embu_ref[...], embi_ref[...]], axis=1)
    uv = jnp.dot(emb, wc_ref[...],
                 preferred_element_type=jnp.float32) + bc_ref[...]
    out_ref[...] = jnp.sum(uv[:, :64] * uv[:, 64:], axis=1, keepdims=True)


def _tc_dense(embu, embi, uW0, ub0, uW1, ub1, uW2, ub2, iW0, ib0, iW1, ib1, iW2, ib2):
    full = lambda shape: pl.BlockSpec(shape, lambda i: (0,) * len(shape))
    return pl.pallas_call(
        _tc_body,
        grid=(B // BB,),
        in_specs=[
            pl.BlockSpec((BB, 13 * 128), lambda i: (i, 0)),
            pl.BlockSpec((BB, 13 * 128), lambda i: (i, 0)),
            full((416, 256)), full((256, 128)), full((128, 64)),
            full((256,)), full((128,)), full((64,)),
            full((416, 256)), full((256, 128)), full((128, 64)),
            full((256,)), full((128,)), full((64,)),
        ],
        out_specs=pl.BlockSpec((BB, 1), lambda i: (i, 0)),
        out_shape=jax.ShapeDtypeStruct((B, 1), jnp.float32),
        scratch_shapes=[
            pltpu.VMEM((ECOLS, 128), jnp.float32),
            pltpu.VMEM((1, 128), jnp.float32),
        ],
    )(embu, embi, uW0, uW1, uW2, ub0, ub1, ub2, iW0, iW1, iW2, ib0, ib1, ib2)


def kernel(X, user_tables, item_tables, uW0, ub0, uW1, ub1, uW2, ub2,
           iW0, ib0, iW1, ib1, iW2, ib2):
    put = _relayout(user_tables)
    pit = _relayout(item_tables)

    xt = X.T.astype(jnp.int32)
    offs = jnp.array([(f // 4) * FP for f in range(13)],
                     dtype=jnp.int32)[:, None]
    xq_u = (xt[:13] + offs).reshape(13, NW, CH, IDX_CH)
    xq_i = (xt[13:] + offs).reshape(13, NW, CH, IDX_CH)

    embu = _sc_gather(xq_u, put)
    embi = _sc_gather(xq_i, pit)
    return _tc_dense(embu, embi, uW0, ub0, uW1, ub1, uW2, ub2,
                     iW0, ib0, iW1, ib1, iW2, ib2)


# speedup vs baseline: 1.0052x; 1.0052x over previous
import functools

import jax
import jax.numpy as jnp
from jax import lax
from jax.experimental import pallas as pl
from jax.experimental.pallas import tpu as pltpu
import jax.experimental.pallas.tpu_sc as plsc

B, F, V, D = 16384, 26, 100000, 32
NC, NS = 2, 16
NW = NC * NS
BPW = B // NW
IDX_CH = 128
CH = BPW // IDX_CH
VB = 12800
NVB = (V + VB - 1) // VB
FP = NVB * VB
ECOLS = F * 128
UCOLS = 13 * D
BB = 1024


def _relayout_body(x0_ref, x1_ref, x2_ref, x3_ref, o_ref):
    xs = jnp.concatenate(
        [x0_ref[0], x1_ref[0], x2_ref[0], x3_ref[0]], axis=0)
    r = lax.broadcasted_iota(jnp.int32, (128, 128), 0)
    c = lax.broadcasted_iota(jnp.int32, (128, 128), 1)
    eye = jnp.where(r == c, 1.0, 0.0).astype(jnp.float32)
    o_ref[...] = lax.dot_general(
        xs, eye, (((0,), (0,)), ((), ())),
        preferred_element_type=jnp.float32)


def _relayout(tables):
    tt = jnp.transpose(tables, (0, 2, 1))
    mk = lambda t: pl.BlockSpec(
        (1, D, VB), lambda g, v, t=t: (jnp.minimum(4 * g + t, 12), 0, v))
    return pl.pallas_call(
        _relayout_body,
        grid=(4, NVB),
        in_specs=[mk(0), mk(1), mk(2), mk(3)],
        out_specs=pl.BlockSpec((VB, 128), lambda g, v: (g * NVB + v, 0)),
        out_shape=jax.ShapeDtypeStruct((4 * FP, 128), jnp.float32),
        compiler_params=pltpu.CompilerParams(
            fuse_transposed_lhs_in_matmul=True),
    )(tt, tt, tt, tt)


def _sc_gather(xq, put, pit):
    mesh = plsc.VectorSubcoreMesh(
        core_axis_name="c", subcore_axis_name="s",
        num_cores=NC, num_subcores=NS)

    NBUF = 4
    LAG = 2

    @functools.partial(
        pl.kernel,
        out_type=jax.ShapeDtypeStruct((B, ECOLS), jnp.float32),
        mesh=mesh,
        scratch_types=[
            pltpu.VMEM((2, CH, IDX_CH), jnp.int32),
            pltpu.VMEM((NBUF, IDX_CH, 128), jnp.float32),
            pltpu.SemaphoreType.DMA,
            pltpu.SemaphoreType.DMA,
        ],
    )
    def k(xq_hbm, put_hbm, pit_hbm, out_hbm, idx_v, rows_v, gsem, wsem):
        wid = lax.axis_index("s") * NC + lax.axis_index("c")
        base = wid * BPW
        nchk = F * CH
        gh = [None] * nchk
        wh = [None] * nchk

        def start_gather(kk):
            t, j = kk // CH, kk % CH
            if j == 0:
                pltpu.sync_copy(xq_hbm.at[t, wid], idx_v.at[t % 2])
            if kk >= NBUF:
                wh[kk - NBUF].wait()
            tab = put_hbm if t < 13 else pit_hbm
            gh[kk] = pltpu.async_copy(
                tab.at[idx_v.at[t % 2, j]], rows_v.at[kk % NBUF], gsem)

        def start_write(kk):
            t, j = kk // CH, kk % CH
            gh[kk].wait()
            wh[kk] = pltpu.async_copy(
                rows_v.at[kk % NBUF],
                out_hbm.at[pl.ds(base + j * IDX_CH, IDX_CH),
                           pl.ds(128 * t, 128)], wsem)

        for kk in range(nchk):
            start_gather(kk)
            if kk >= LAG:
                start_write(kk - LAG)
        for kk in range(nchk - LAG, nchk):
            start_write(kk)
        for kk in range(nchk - NBUF, nchk):
            wh[kk].wait()

    return k(xq, put, pit)


def _tc_body(emb_ref, uW0_ref, uW1_ref, uW2_ref, ub0_ref, ub1_ref, ub2_ref,
             iW0_ref, iW1_ref, iW2_ref, ib0_ref, ib1_ref, ib2_ref,
             out_ref, wc_ref, bc_ref):
    @pl.when(pl.program_id(0) == 0)
    def _():
        f32 = jnp.float32
        wu = jnp.dot(uW0_ref[...], jnp.dot(uW1_ref[...], uW2_ref[...],
                                           preferred_element_type=f32),
                     preferred_element_type=f32)
        wv = jnp.dot(iW0_ref[...], jnp.dot(iW1_ref[...], iW2_ref[...],
                                           preferred_element_type=f32),
                     preferred_element_type=f32)
        wc_ref[...] = jnp.zeros((ECOLS, 128), f32)
        for t in range(13):
            ro = 32 * (t % 4)
            wc_ref[128 * t + ro:128 * t + ro + D, 0:64] = wu[D * t:D * t + D, :]
            wc_ref[128 * (13 + t) + ro:128 * (13 + t) + ro + D, 64:128] = \
                wv[D * t:D * t + D, :]
        bu = jnp.dot(jnp.dot(ub0_ref[...].reshape(1, 256), uW1_ref[...],
                             preferred_element_type=f32)
                     + ub1_ref[...].reshape(1, 128),
                     uW2_ref[...], preferred_element_type=f32) \
            + ub2_ref[...].reshape(1, 64)
        bv = jnp.dot(jnp.dot(ib0_ref[...].reshape(1, 256), iW1_ref[...],
                             preferred_element_type=f32)
                     + ib1_ref[...].reshape(1, 128),
                     iW2_ref[...], preferred_element_type=f32) \
            + ib2_ref[...].reshape(1, 64)
        bc_ref[0:1, 0:64] = bu
        bc_ref[0:1, 64:128] = bv

    uv = jnp.dot(emb_ref[...], wc_ref[...],
                 preferred_element_type=jnp.float32) + bc_ref[...]
    out_ref[...] = jnp.sum(uv[:, :64] * uv[:, 64:], axis=1, keepdims=True)


def _tc_dense(emb, uW0, ub0, uW1, ub1, uW2, ub2, iW0, ib0, iW1, ib1, iW2, ib2):
    full = lambda shape: pl.BlockSpec(shape, lambda i: (0,) * len(shape))
    return pl.pallas_call(
        _tc_body,
        grid=(B // BB,),
        in_specs=[
            pl.BlockSpec((BB, ECOLS), lambda i: (i, 0)),
            full((416, 256)), full((256, 128)), full((128, 64)),
            full((256,)), full((128,)), full((64,)),
            full((416, 256)), full((256, 128)), full((128, 64)),
            full((256,)), full((128,)), full((64,)),
        ],
        out_specs=pl.BlockSpec((BB, 1), lambda i: (i, 0)),
        out_shape=jax.ShapeDtypeStruct((B, 1), jnp.float32),
        scratch_shapes=[
            pltpu.VMEM((ECOLS, 128), jnp.float32),
            pltpu.VMEM((1, 128), jnp.float32),
        ],
    )(emb, uW0, uW1, uW2, ub0, ub1, ub2, iW0, iW1, iW2, ib0, ib1, ib2)


def kernel(X, user_tables, item_tables, uW0, ub0, uW1, ub1, uW2, ub2,
           iW0, ib0, iW1, ib1, iW2, ib2):
    put = _relayout(user_tables)
    pit = _relayout(item_tables)

    xt = X.T.astype(jnp.int32)
    offs = jnp.array([(f // 4) * FP for f in range(13)] * 2,
                     dtype=jnp.int32)[:, None]
    xq = (xt + offs).reshape(F, NW, CH, IDX_CH)

    emb = _sc_gather(xq, put, pit)
    return _tc_dense(emb, uW0, ub0, uW1, ub1, uW2, ub2,
                     iW0, ib0, iW1, ib1, iW2, ib2)
